# baseline (speedup 1.0000x reference)
import jax
import jax.numpy as jnp
from jax.experimental import pallas as pl

B = 4096
N = 50
D = 128
H1 = 256
OUT = 128
R = 3

BB = 64
NP = 56
T = BB * NP


def _lk(x):
    return jnp.maximum(x, x * jnp.asarray(0.01, x.dtype))


def _dot_t(x, w, out_dtype):
    return jax.lax.dot_general(x, w, (((1,), (1,)), ((), ())),
                               preferred_element_type=out_dtype)


def _fused_body(u_ref, i_ref, a_ref, o_ref, s_ref,
                uw0_ref, ub0_ref, uw1_ref, ub1_ref,
                aw0_ref, ab0_ref, aw1_ref, ab1_ref,
                pred_ref):
    f32 = jnp.float32
    bf16 = jnp.bfloat16

    u = u_ref[...].astype(bf16)
    i = i_ref[...].astype(bf16)
    h_ui = _lk(
        _dot_t(u, uw0_ref[:, :D], f32)
        + _dot_t(i, uw0_ref[:, D:], f32)
        + ub0_ref[...]
    )
    ue = _lk(_dot_t(h_ui.astype(bf16), uw1_ref[...], f32) + ub1_ref[...])
    ue_b = ue[:, None, :]

    zpad = jnp.zeros((BB, NP - N, D), dtype=f32)
    xa = jnp.concatenate([a_ref[...], zpad], axis=1).reshape(T, D)
    xo = jnp.concatenate([o_ref[...], zpad], axis=1).reshape(T, D)
    x = jnp.concatenate([xa, xo], axis=1).astype(bf16)

    h_all = _dot_t(x, aw0_ref[...], f32) + ab0_ref[...]
    h_all = _lk(h_all.astype(bf16))

    s = s_ref[...]
    pred = jnp.zeros((BB, N), dtype=f32)
    for r in range(R):
        h_r = h_all[:, r * H1:(r + 1) * H1]
        out_r = _lk(_dot_t(h_r, aw1_ref[r], f32) + ab1_ref[r])
        p_r = jnp.sum(out_r.reshape(BB, NP, OUT) * ue_b, axis=-1)
        pred = pred + jnp.where(s == r, p_r[:, :N], 0.0)
    pred_ref[...] = pred


def kernel(u_emb, i_emb, a_emb, o_emb, s, ui_W0, ui_b0, ui_W1, ui_b1,
           ao_W0, ao_b0, ao_W1, ao_b1):
    bf16 = jnp.bfloat16
    uw0 = ui_W0.astype(bf16)
    uw1 = ui_W1.astype(bf16)
    aw0 = ao_W0.reshape(R * H1, 2 * D).astype(bf16)
    ab0 = ao_b0.reshape(R * H1)
    aw1 = ao_W1.astype(bf16)
    s32 = s.astype(jnp.int32)

    grid = (B // BB,)

    def const(shape):
        nd = len(shape)
        return pl.BlockSpec(shape, lambda i: (0,) * nd)

    out = pl.pallas_call(
        _fused_body,
        grid=grid,
        in_specs=[
            pl.BlockSpec((BB, D), lambda i: (i, 0)),
            pl.BlockSpec((BB, D), lambda i: (i, 0)),
            pl.BlockSpec((BB, N, D), lambda i: (i, 0, 0)),
            pl.BlockSpec((BB, N, D), lambda i: (i, 0, 0)),
            pl.BlockSpec((BB, N), lambda i: (i, 0)),
            const((H1, 2 * D)), const((H1,)),
            const((OUT, H1)), const((OUT,)),
            const((R * H1, 2 * D)), const((R * H1,)),
            const((R, OUT, H1)), const((R, OUT)),
        ],
        out_specs=pl.BlockSpec((BB, N), lambda i: (i, 0)),
        out_shape=jax.ShapeDtypeStruct((B, N), jnp.float32),
    )(u_emb, i_emb, a_emb, o_emb, s32,
      uw0, ui_b0, uw1, ui_b1,
      aw0, ab0, aw1, ao_b1)
    return out
